# Initial kernel scaffold; baseline (speedup 1.0000x reference)
import jax
import jax.numpy as jnp
from jax.experimental import pallas as pl


def kernel(x):
    raise NotImplementedError("write your pallas kernel here")


# speedup vs baseline: 253.4430x; 253.4430x over previous
import jax
import jax.numpy as jnp
from jax.experimental import pallas as pl

_K = 10000
_INT_MIN = -2147483648


def _kwta_kernel(x_ref, o_ref):
    iu = jax.lax.bitcast_convert_type(x_ref[...], jnp.int32)
    key = iu ^ ((iu >> 31) & jnp.int32(0x7FFFFFFF))
    key = jnp.where(key == jnp.int32(-1), jnp.int32(0), key)

    sign_cnt = jnp.sum((key >= jnp.int32(0)).astype(jnp.int32))
    prefix0 = jnp.where(sign_cnt >= _K, jnp.int32(0), jnp.int32(_INT_MIN))

    def body(i, prefix):
        bit = jnp.int32(30) - i
        trial = prefix | (jnp.int32(1) << bit)
        cnt = jnp.sum((key >= trial).astype(jnp.int32))
        return jnp.where(cnt >= _K, trial, prefix)

    t = jax.lax.fori_loop(0, 31, body, prefix0)
    o_ref[...] = (key >= t).astype(jnp.float32)


@jax.jit
def kernel(x):
    return pl.pallas_call(
        _kwta_kernel,
        out_shape=jax.ShapeDtypeStruct(x.shape, x.dtype),
    )(x)
